# baseline (speedup 1.0000x reference)
import functools

import jax
import jax.numpy as jnp
from jax import lax
from jax.experimental import pallas as pl
from jax.experimental.pallas import tpu as pltpu
from jax.experimental.pallas import tpu_sc as plsc

NC = 2
NS = 16
L = 16
K = 128
DW = 16
NB = 4
NW = NC * NS
BM = 2048


def _sc_mesh():
    return plsc.VectorSubcoreMesh(
        core_axis_name="c", subcore_axis_name="s", num_cores=NC, num_subcores=NS
    )


_SC_PARAMS = pltpu.CompilerParams(use_tc_tiling_on_sc=False)


def _make_deg_kernel(n0, npad):
    rows_per_tile = npad // NS

    @functools.partial(
        pl.kernel,
        mesh=_sc_mesh(),
        compiler_params=_SC_PARAMS,
        out_type=jax.ShapeDtypeStruct((NC, npad, DW), jnp.float32),
        scratch_types=[
            pltpu.VMEM((n0, K), jnp.int32),
            pltpu.VMEM((K, DW), jnp.float32),
            pltpu.VMEM((K, DW), jnp.float32),
            pltpu.VMEM_SHARED((npad, DW), jnp.float32),
        ],
    )
    def deg_kernel(dst_hbm, out_hbm, didx_all, ones_v, zeros_v, accum):
        c = lax.axis_index("c")
        s = lax.axis_index("s")
        wid = c * NS + s
        base = s * rows_per_tile

        pltpu.sync_copy(dst_hbm.at[wid], didx_all)

        def fill_row(i, carry):
            ones_v[i, pl.ds(0, L)] = jnp.full((L,), 1.0, jnp.float32)
            zeros_v[i, pl.ds(0, L)] = jnp.zeros((L,), jnp.float32)
            return carry

        lax.fori_loop(0, K, fill_row, 0)

        def zero_stripe(t, carry):
            pltpu.sync_copy(zeros_v, accum.at[pl.ds(base + t * K, K)])
            return carry

        lax.fori_loop(0, rows_per_tile // K, zero_stripe, 0)
        plsc.subcore_barrier()

        def chunk(j, carry):
            pltpu.sync_copy(ones_v, accum.at[didx_all.at[j]], add=True)
            return carry

        lax.fori_loop(0, n0, chunk, 0)
        plsc.subcore_barrier()

        def writeback(t, carry):
            pltpu.sync_copy(
                accum.at[pl.ds(base + t * K, K)],
                out_hbm.at[c, pl.ds(base + t * K, K)],
            )
            return carry

        lax.fori_loop(0, rows_per_tile // K, writeback, 0)

    return deg_kernel


GC = 8


def _make_spmm_kernel(d, n0, npad):
    rows_per_tile = npad // NS
    n_groups = n0 // GC
    assert n0 % (2 * GC) == 0 and n_groups >= 2 and GC % NB == 0

    @functools.partial(
        pl.kernel,
        mesh=_sc_mesh(),
        compiler_params=_SC_PARAMS,
        out_type=jax.ShapeDtypeStruct((NC, npad, d), jnp.float32),
        scratch_types=[
            [pltpu.VMEM((GC, 2, K), jnp.int32) for _ in range(2)],
            [pltpu.VMEM((K, d), jnp.float32) for _ in range(NB)],
            pltpu.VMEM((K, d), jnp.float32),
            pltpu.VMEM_SHARED((npad, d), jnp.float32),
            [pltpu.SemaphoreType.DMA for _ in range(2)],
            [pltpu.SemaphoreType.DMA for _ in range(NB)],
        ],
    )
    def spmm_kernel(hp_hbm, idx_hbm, out_hbm,
                    slots, rows, zeros_v, accum, semI, semG):
        c = lax.axis_index("c")
        s = lax.axis_index("s")
        wid = c * NS + s
        base = s * rows_per_tile

        def wait_rows(b):
            pltpu.make_async_copy(hp_hbm.at[pl.ds(0, K)], rows[b],
                                  semG[b]).wait()

        def wait_slot(p):
            pltpu.make_async_copy(idx_hbm.at[wid, 0], slots[p], semI[p]).wait()

        def emit_group(gX, j0, p, has_next, tail):
            slot, other = slots[p], slots[1 - p]
            if has_next:
                pltpu.async_copy(idx_hbm.at[wid, gX + 1], other, semI[1 - p])
            for u in range(GC):
                b = u % NB
                if u == GC - NB and has_next:
                    wait_slot(1 - p)
                wait_rows(b)
                pltpu.sync_copy(rows[b], accum.at[slot.at[u, 1]], add=True)
                if not tail or j0 + u + NB < n0:
                    if u < GC - NB:
                        nslot, nu = slot, u + NB
                    else:
                        nslot, nu = other, u + NB - GC
                    pltpu.async_copy(hp_hbm.at[nslot.at[nu, 0]], rows[b],
                                     semG[b])

        if True:
            def zero_row(i, carry):
                def zero_col(t, inner):
                    zeros_v[i, pl.ds(t * L, L)] = jnp.zeros((L,), jnp.float32)
                    return inner
                return lax.fori_loop(0, d // L, zero_col, carry)

            lax.fori_loop(0, K, zero_row, 0)

            def zero_stripe(t, carry):
                pltpu.sync_copy(zeros_v, accum.at[pl.ds(base + t * K, K)])
                return carry

            lax.fori_loop(0, rows_per_tile // K, zero_stripe, 0)
            plsc.subcore_barrier()

            pltpu.sync_copy(idx_hbm.at[wid, 0], slots[0])
            for u in range(NB):
                pltpu.async_copy(hp_hbm.at[slots[0].at[u, 0]], rows[u],
                                 semG[u])

            def pair(g2, carry):
                a = 2 * g2
                emit_group(a, a * GC, 0, True, False)
                emit_group(a + 1, (a + 1) * GC, 1, True, False)
                return carry

            lax.fori_loop(0, n_groups // 2 - 1, pair, 0)
            gA = n_groups - 2
            emit_group(gA, gA * GC, 0, True, True)
            gB = n_groups - 1
            emit_group(gB, gB * GC, 1, False, True)
            plsc.subcore_barrier()

            def writeback(t, carry):
                pltpu.sync_copy(
                    accum.at[pl.ds(base + t * K, K)],
                    out_hbm.at[c, pl.ds(base + t * K, K)],
                )
                return carry

            lax.fori_loop(0, rows_per_tile // K, writeback, 0)

    return spmm_kernel


def _dot(a, b):
    return jax.lax.dot_general(
        a, b, (((1,), (0,)), ((), ())),
        precision=jax.lax.Precision.HIGHEST,
        preferred_element_type=jnp.float32,
    )


def _tc_first(x_pad, W1, deg, npad, d_in, d_out):
    grid = (npad // BM,)

    def body(x_ref, w_ref, deg_ref, h_ref, dv_ref):
        dinv = lax.rsqrt(deg_ref[0, :, 0] + deg_ref[1, :, 0] + 1.0)[:, None]
        dv_ref[...] = jnp.broadcast_to(dinv, (BM, 8))
        h_ref[...] = _dot(x_ref[...], w_ref[...]) * dinv

    return pl.pallas_call(
        body,
        grid=grid,
        in_specs=[
            pl.BlockSpec((BM, d_in), lambda i: (i, 0)),
            pl.BlockSpec((d_in, d_out), lambda i: (0, 0)),
            pl.BlockSpec((NC, BM, DW), lambda i: (0, i, 0)),
        ],
        out_specs=[
            pl.BlockSpec((BM, d_out), lambda i: (i, 0)),
            pl.BlockSpec((BM, 8), lambda i: (i, 0)),
        ],
        out_shape=[
            jax.ShapeDtypeStruct((npad, d_out), jnp.float32),
            jax.ShapeDtypeStruct((npad, 8), jnp.float32),
        ],
    )(x_pad, W1, deg)


def _tc_mid(S, hp, dinv_col, b, W, npad, d_in, d_out):
    grid = (npad // BM,)
    nout = -(-d_out // 64)

    def body(s_ref, h_ref, dv_ref, b_ref, w_ref, *o_refs):
        dv = dv_ref[:, :1]
        acc = s_ref[0] + s_ref[1] + h_ref[...]
        xv = dv * acc + b_ref[...]
        xv = jnp.where(xv >= 0, xv, 0.2 * xv)
        r = _dot(xv, w_ref[...]) * dv
        for t, o_ref in enumerate(o_refs):
            o_ref[...] = r[:, t * 64:(t + 1) * 64]

    return pl.pallas_call(
        body,
        grid=grid,
        in_specs=[
            pl.BlockSpec((NC, BM, d_in), lambda i: (0, i, 0)),
            pl.BlockSpec((BM, d_in), lambda i: (i, 0)),
            pl.BlockSpec((BM, 8), lambda i: (i, 0)),
            pl.BlockSpec((1, d_in), lambda i: (0, 0)),
            pl.BlockSpec((d_in, d_out), lambda i: (0, 0)),
        ],
        out_specs=[pl.BlockSpec((BM, 64), lambda i: (i, 0))] * nout,
        out_shape=[jax.ShapeDtypeStruct((npad, 64), jnp.float32)] * nout,
    )(S, hp, dinv_col, b, W)


def _tc_last(Sa, Sb, hpa, hpb, dinv_col, b, npad, d):
    grid = (npad // BM,)

    def body(sa_ref, sb_ref, ha_ref, hb_ref, dv_ref, b_ref, o_ref):
        dv = dv_ref[:, :1]
        acc_a = sa_ref[0] + sa_ref[1] + ha_ref[...]
        acc_b = sb_ref[0] + sb_ref[1] + hb_ref[...]
        acc = jnp.concatenate([acc_a, acc_b], axis=1)
        o_ref[...] = jnp.tanh(dv * acc + b_ref[...])

    return pl.pallas_call(
        body,
        grid=grid,
        in_specs=[
            pl.BlockSpec((NC, BM, 64), lambda i: (0, i, 0)),
            pl.BlockSpec((NC, BM, 64), lambda i: (0, i, 0)),
            pl.BlockSpec((BM, 64), lambda i: (i, 0)),
            pl.BlockSpec((BM, 64), lambda i: (i, 0)),
            pl.BlockSpec((BM, 8), lambda i: (i, 0)),
            pl.BlockSpec((1, d), lambda i: (0, 0)),
        ],
        out_specs=pl.BlockSpec((BM, d), lambda i: (i, 0)),
        out_shape=jax.ShapeDtypeStruct((npad, d), jnp.float32),
    )(Sa, Sb, hpa, hpb, dinv_col, b)


def kernel(x, edge_index, W1, b1, W2, b2, W3, b3):
    n, d_in = x.shape
    d_hid = W1.shape[1]
    e = edge_index.shape[1]

    npad = -(-n // (NS * K)) * (NS * K)
    n0 = -(-(-(-e // (NW * K))) // (2 * GC)) * (2 * GC)
    e_pad = NW * n0 * K

    ei = edge_index.astype(jnp.int32)
    pad = n + jnp.arange(e_pad - e, dtype=jnp.int32) % (npad - n)
    src3 = jnp.concatenate([ei[0], pad]).reshape(NW, n0, K)
    dst3 = jnp.concatenate([ei[1], pad]).reshape(NW, n0, K)
    idx5 = jnp.stack([src3, dst3], axis=2).reshape(NW, n0 // GC, GC, 2, K)
    x_pad = jnp.pad(x, ((0, npad - n), (0, 0)))

    spmm64 = _make_spmm_kernel(d_hid, n0, npad)
    deg = _make_deg_kernel(n0, npad)(dst3)
    h1p, dinv_col = _tc_first(x_pad, W1, deg, npad, d_in, d_hid)
    S1 = spmm64(h1p, idx5)
    (h2p,) = _tc_mid(S1, h1p, dinv_col, b1.reshape(1, -1), W2,
                     npad, d_hid, d_hid)
    S2 = spmm64(h2p, idx5)
    h3pa, h3pb = _tc_mid(S2, h2p, dinv_col, b2.reshape(1, -1), W3,
                         npad, d_hid, d_in)
    S3a = spmm64(h3pa, idx5)
    S3b = spmm64(h3pb, idx5)
    out = _tc_last(S3a, S3b, h3pa, h3pb, dinv_col, b3.reshape(1, -1),
                   npad, d_in)
    return out[:n]


# speedup vs baseline: 1.0029x; 1.0029x over previous
import functools

import jax
import jax.numpy as jnp
from jax import lax
from jax.experimental import pallas as pl
from jax.experimental.pallas import tpu as pltpu
from jax.experimental.pallas import tpu_sc as plsc

NC = 2
NS = 16
L = 16
K = 128
DW = 16
NB = 4
NW = NC * NS
BM = 2048


def _sc_mesh():
    return plsc.VectorSubcoreMesh(
        core_axis_name="c", subcore_axis_name="s", num_cores=NC, num_subcores=NS
    )


_SC_PARAMS = pltpu.CompilerParams(use_tc_tiling_on_sc=False)


def _make_deg_kernel(n0, npad):
    rows_per_tile = npad // NS

    @functools.partial(
        pl.kernel,
        mesh=_sc_mesh(),
        compiler_params=_SC_PARAMS,
        out_type=jax.ShapeDtypeStruct((NC, npad, DW), jnp.float32),
        scratch_types=[
            pltpu.VMEM((n0 // GC, GC, 2, K), jnp.int32),
            pltpu.VMEM((K, DW), jnp.float32),
            pltpu.VMEM((K, DW), jnp.float32),
            pltpu.VMEM_SHARED((npad, DW), jnp.float32),
        ],
    )
    def deg_kernel(dst_hbm, out_hbm, didx_all, ones_v, zeros_v, accum):
        c = lax.axis_index("c")
        s = lax.axis_index("s")
        wid = c * NS + s
        base = s * rows_per_tile

        pltpu.sync_copy(dst_hbm.at[wid], didx_all)

        def fill_row(i, carry):
            ones_v[i, pl.ds(0, L)] = jnp.full((L,), 1.0, jnp.float32)
            zeros_v[i, pl.ds(0, L)] = jnp.zeros((L,), jnp.float32)
            return carry

        lax.fori_loop(0, K, fill_row, 0)

        def zero_stripe(t, carry):
            pltpu.sync_copy(zeros_v, accum.at[pl.ds(base + t * K, K)])
            return carry

        lax.fori_loop(0, rows_per_tile // K, zero_stripe, 0)
        plsc.subcore_barrier()

        def chunk(j, carry):
            def one(u, inner):
                pltpu.sync_copy(ones_v, accum.at[didx_all.at[j, u, 1]],
                                add=True)
                return inner
            return lax.fori_loop(0, GC, one, carry)

        lax.fori_loop(0, n0 // GC, chunk, 0)
        plsc.subcore_barrier()

        def writeback(t, carry):
            pltpu.sync_copy(
                accum.at[pl.ds(base + t * K, K)],
                out_hbm.at[c, pl.ds(base + t * K, K)],
            )
            return carry

        lax.fori_loop(0, rows_per_tile // K, writeback, 0)

    return deg_kernel


GC = 8


def _make_spmm_kernel(d, n0, npad):
    rows_per_tile = npad // NS
    n_groups = n0 // GC
    assert n0 % (2 * GC) == 0 and n_groups >= 2 and GC % NB == 0

    @functools.partial(
        pl.kernel,
        mesh=_sc_mesh(),
        compiler_params=_SC_PARAMS,
        out_type=jax.ShapeDtypeStruct((NC, npad, d), jnp.float32),
        scratch_types=[
            [pltpu.VMEM((GC, 2, K), jnp.int32) for _ in range(2)],
            [pltpu.VMEM((K, d), jnp.float32) for _ in range(NB)],
            pltpu.VMEM((K, d), jnp.float32),
            pltpu.VMEM_SHARED((npad, d), jnp.float32),
            [pltpu.SemaphoreType.DMA for _ in range(2)],
            [pltpu.SemaphoreType.DMA for _ in range(NB)],
        ],
    )
    def spmm_kernel(hp_hbm, idx_hbm, out_hbm,
                    slots, rows, zeros_v, accum, semI, semG):
        c = lax.axis_index("c")
        s = lax.axis_index("s")
        wid = c * NS + s
        base = s * rows_per_tile

        def wait_rows(b):
            pltpu.make_async_copy(hp_hbm.at[pl.ds(0, K)], rows[b],
                                  semG[b]).wait()

        def wait_slot(p):
            pltpu.make_async_copy(idx_hbm.at[wid, 0], slots[p], semI[p]).wait()

        def emit_group(gX, j0, p, has_next, tail):
            slot, other = slots[p], slots[1 - p]
            if has_next:
                pltpu.async_copy(idx_hbm.at[wid, gX + 1], other, semI[1 - p])
            for u in range(GC):
                b = u % NB
                if u == GC - NB and has_next:
                    wait_slot(1 - p)
                wait_rows(b)
                pltpu.sync_copy(rows[b], accum.at[slot.at[u, 1]], add=True)
                if not tail or j0 + u + NB < n0:
                    if u < GC - NB:
                        nslot, nu = slot, u + NB
                    else:
                        nslot, nu = other, u + NB - GC
                    pltpu.async_copy(hp_hbm.at[nslot.at[nu, 0]], rows[b],
                                     semG[b])

        if True:
            def zero_row(i, carry):
                def zero_col(t, inner):
                    zeros_v[i, pl.ds(t * L, L)] = jnp.zeros((L,), jnp.float32)
                    return inner
                return lax.fori_loop(0, d // L, zero_col, carry)

            lax.fori_loop(0, K, zero_row, 0)

            def zero_stripe(t, carry):
                pltpu.sync_copy(zeros_v, accum.at[pl.ds(base + t * K, K)])
                return carry

            lax.fori_loop(0, rows_per_tile // K, zero_stripe, 0)
            plsc.subcore_barrier()

            pltpu.sync_copy(idx_hbm.at[wid, 0], slots[0])
            for u in range(NB):
                pltpu.async_copy(hp_hbm.at[slots[0].at[u, 0]], rows[u],
                                 semG[u])

            def pair(g2, carry):
                a = 2 * g2
                emit_group(a, a * GC, 0, True, False)
                emit_group(a + 1, (a + 1) * GC, 1, True, False)
                return carry

            lax.fori_loop(0, n_groups // 2 - 1, pair, 0)
            gA = n_groups - 2
            emit_group(gA, gA * GC, 0, True, True)
            gB = n_groups - 1
            emit_group(gB, gB * GC, 1, False, True)
            plsc.subcore_barrier()

            def writeback(t, carry):
                pltpu.sync_copy(
                    accum.at[pl.ds(base + t * K, K)],
                    out_hbm.at[c, pl.ds(base + t * K, K)],
                )
                return carry

            lax.fori_loop(0, rows_per_tile // K, writeback, 0)

    return spmm_kernel


def _dot(a, b):
    return jax.lax.dot_general(
        a, b, (((1,), (0,)), ((), ())),
        precision=jax.lax.Precision.HIGHEST,
        preferred_element_type=jnp.float32,
    )


def _tc_first(x_pad, W1, deg, npad, d_in, d_out):
    grid = (npad // BM,)

    def body(x_ref, w_ref, deg_ref, h_ref, dv_ref):
        dinv = lax.rsqrt(deg_ref[0, :, 0] + deg_ref[1, :, 0] + 1.0)[:, None]
        dv_ref[...] = jnp.broadcast_to(dinv, (BM, 128))
        h_ref[...] = _dot(x_ref[...], w_ref[...]) * dinv

    return pl.pallas_call(
        body,
        grid=grid,
        in_specs=[
            pl.BlockSpec((BM, d_in), lambda i: (i, 0)),
            pl.BlockSpec((d_in, d_out), lambda i: (0, 0)),
            pl.BlockSpec((NC, BM, DW), lambda i: (0, i, 0)),
        ],
        out_specs=[
            pl.BlockSpec((BM, d_out), lambda i: (i, 0)),
            pl.BlockSpec((BM, 128), lambda i: (i, 0)),
        ],
        out_shape=[
            jax.ShapeDtypeStruct((npad, d_out), jnp.float32),
            jax.ShapeDtypeStruct((npad, 128), jnp.float32),
        ],
    )(x_pad, W1, deg)


def _tc_mid(S, hp, dinv_col, b, W, npad, d_in, d_out):
    grid = (npad // BM,)
    nout = -(-d_out // 64)

    def body(s_ref, h_ref, dv_ref, b_ref, w_ref, *o_refs):
        dv = dv_ref[:, :1]
        acc = s_ref[0] + s_ref[1] + h_ref[...]
        xv = dv * acc + b_ref[...]
        xv = jnp.where(xv >= 0, xv, 0.2 * xv)
        r = _dot(xv, w_ref[...]) * dv
        for t, o_ref in enumerate(o_refs):
            o_ref[...] = r[:, t * 64:(t + 1) * 64]

    return pl.pallas_call(
        body,
        grid=grid,
        in_specs=[
            pl.BlockSpec((NC, BM, d_in), lambda i: (0, i, 0)),
            pl.BlockSpec((BM, d_in), lambda i: (i, 0)),
            pl.BlockSpec((BM, 128), lambda i: (i, 0)),
            pl.BlockSpec((1, d_in), lambda i: (0, 0)),
            pl.BlockSpec((d_in, d_out), lambda i: (0, 0)),
        ],
        out_specs=[pl.BlockSpec((BM, 64), lambda i: (i, 0))] * nout,
        out_shape=[jax.ShapeDtypeStruct((npad, 64), jnp.float32)] * nout,
    )(S, hp, dinv_col, b, W)


def _tc_last(Sa, Sb, hpa, hpb, dinv_col, b, npad, nout, d):
    grid = (npad // BM,)

    def body(sa_ref, sb_ref, ha_ref, hb_ref, dv_ref, b_ref, o_ref):
        dv = dv_ref[:, :1]
        acc_a = sa_ref[0] + sa_ref[1] + ha_ref[...]
        acc_b = sb_ref[0] + sb_ref[1] + hb_ref[...]
        acc = jnp.concatenate([acc_a, acc_b], axis=1)
        o_ref[...] = jnp.tanh(dv * acc + b_ref[...])

    return pl.pallas_call(
        body,
        grid=grid,
        in_specs=[
            pl.BlockSpec((NC, BM, 64), lambda i: (0, i, 0)),
            pl.BlockSpec((NC, BM, 64), lambda i: (0, i, 0)),
            pl.BlockSpec((BM, 64), lambda i: (i, 0)),
            pl.BlockSpec((BM, 64), lambda i: (i, 0)),
            pl.BlockSpec((BM, 128), lambda i: (i, 0)),
            pl.BlockSpec((1, d), lambda i: (0, 0)),
        ],
        out_specs=pl.BlockSpec((BM, d), lambda i: (i, 0)),
        out_shape=jax.ShapeDtypeStruct((nout, d), jnp.float32),
    )(Sa, Sb, hpa, hpb, dinv_col, b)


def kernel(x, edge_index, W1, b1, W2, b2, W3, b3):
    n, d_in = x.shape
    d_hid = W1.shape[1]
    e = edge_index.shape[1]

    npad = -(-n // (NS * K)) * (NS * K)
    n0 = -(-(-(-e // (NW * K))) // (2 * GC)) * (2 * GC)
    e_pad = NW * n0 * K

    ei = edge_index.astype(jnp.int32)
    pad = n + jnp.arange(e_pad - e, dtype=jnp.int32) % (npad - n)
    src3 = jnp.concatenate([ei[0], pad]).reshape(NW, n0, K)
    dst3 = jnp.concatenate([ei[1], pad]).reshape(NW, n0, K)
    idx5 = jnp.stack([src3, dst3], axis=2).reshape(NW, n0 // GC, GC, 2, K)
    x_pad = jnp.pad(x, ((0, npad - n), (0, 0)))

    spmm64 = _make_spmm_kernel(d_hid, n0, npad)
    deg = _make_deg_kernel(n0, npad)(idx5)
    h1p, dinv_col = _tc_first(x_pad, W1, deg, npad, d_in, d_hid)
    S1 = spmm64(h1p, idx5)
    (h2p,) = _tc_mid(S1, h1p, dinv_col, b1.reshape(1, -1), W2,
                     npad, d_hid, d_hid)
    S2 = spmm64(h2p, idx5)
    h3pa, h3pb = _tc_mid(S2, h2p, dinv_col, b2.reshape(1, -1), W3,
                         npad, d_hid, d_in)
    S3a = spmm64(h3pa, idx5)
    S3b = spmm64(h3pb, idx5)
    nout_rows = -(-n // 8) * 8
    out = _tc_last(S3a, S3b, h3pa, h3pb, dinv_col, b3.reshape(1, -1),
                   npad, nout_rows, d_in)
    return out[:n]
